# Initial kernel scaffold; baseline (speedup 1.0000x reference)
import jax
import jax.numpy as jnp
from jax.experimental import pallas as pl


def kernel(physics_x, structural_x, W_struct, b_struct, gamma, beta, W_phys, Wq, Wk, Wv, Wo, atom_to_nuc):
    raise NotImplementedError("write your pallas kernel here")


# speedup vs baseline: 1.6305x; 1.6305x over previous
import jax
import jax.numpy as jnp
from jax.experimental import pallas as pl

_N, _H, _NH, _DH = 2048, 512, 8, 64
_BA = 256
_BQ = 256
_NEG = -1e9
_SCALE = 0.125


def _embed_qkv(px_ref, sx_ref, Ws_ref, bs_ref, g_ref, b_ref, Wp_ref,
               Wq_ref, Wk_ref, Wv_ref,
               h_ref, q_ref, k_ref, v_ref, fq_ref, gk_ref):
    px = px_ref[...]
    sx = sx_ref[...]
    h = jax.lax.dot_general(sx, Ws_ref[...], (((1,), (0,)), ((), ())),
                            preferred_element_type=jnp.float32)
    h = h + bs_ref[...]
    mu = jnp.mean(h, axis=1, keepdims=True)
    var = jnp.mean((h - mu) ** 2, axis=1, keepdims=True)
    h = (h - mu) / jnp.sqrt(var + 1e-5) * g_ref[...] + b_ref[...]
    h = h * jax.nn.sigmoid(h)
    h = h + jax.lax.dot_general(px, Wp_ref[...], (((1,), (0,)), ((), ())),
                                preferred_element_type=jnp.float32)
    h_ref[...] = h
    q_ref[...] = jax.lax.dot_general(h, Wq_ref[...], (((1,), (0,)), ((), ())),
                                     preferred_element_type=jnp.float32)
    k_ref[...] = jax.lax.dot_general(h, Wk_ref[...], (((1,), (0,)), ((), ())),
                                     preferred_element_type=jnp.float32)
    v_ref[...] = jax.lax.dot_general(h, Wv_ref[...], (((1,), (0,)), ((), ())),
                                     preferred_element_type=jnp.float32)
    d = (px[:, 6:7] > 0).astype(jnp.int32)
    a = (px[:, 7:8] > 0).astype(jnp.int32)
    ar = (sx[:, 1:2] > 0).astype(jnp.int32)
    fq_ref[...] = d + 2 * a + 4 * ar
    gk_ref[...] = 2 * d + a + 4 * ar


def _attn(fq_ref, gk_ref, nc_ref, nr_ref, q_ref, k_ref, v_ref, o_ref):
    q = q_ref[...] * _SCALE
    k = k_ref[...]
    v = v_ref[...]
    mask = ((fq_ref[...] & gk_ref[...]) != 0) & (nc_ref[...] != nr_ref[...])
    for hh in range(_NH):
        sl = slice(hh * _DH, (hh + 1) * _DH)
        s = jax.lax.dot_general(q[:, sl], k[:, sl], (((1,), (1,)), ((), ())),
                                preferred_element_type=jnp.float32)
        s = jnp.where(mask, s, _NEG)
        m = jnp.max(s, axis=1, keepdims=True)
        p = jnp.exp(s - m)
        l = jnp.sum(p, axis=1, keepdims=True)
        ctx = jax.lax.dot_general(p, v[:, sl], (((1,), (0,)), ((), ())),
                                  preferred_element_type=jnp.float32)
        o_ref[:, sl] = ctx / l


def _out_proj(h_ref, c_ref, Wo_ref, o_ref):
    o_ref[...] = h_ref[...] + jax.lax.dot_general(
        c_ref[...], Wo_ref[...], (((1,), (0,)), ((), ())),
        preferred_element_type=jnp.float32)


def kernel(physics_x, structural_x, W_struct, b_struct, gamma, beta,
           W_phys, Wq, Wk, Wv, Wo, atom_to_nuc):
    nuc_col = atom_to_nuc.astype(jnp.int32).reshape(_N, 1)

    h, q, k, v, fq, gk = pl.pallas_call(
        _embed_qkv,
        grid=(_N // _BA,),
        in_specs=[
            pl.BlockSpec((_BA, 10), lambda i: (i, 0)),
            pl.BlockSpec((_BA, 4), lambda i: (i, 0)),
            pl.BlockSpec((4, _H), lambda i: (0, 0)),
            pl.BlockSpec((1, _H), lambda i: (0, 0)),
            pl.BlockSpec((1, _H), lambda i: (0, 0)),
            pl.BlockSpec((1, _H), lambda i: (0, 0)),
            pl.BlockSpec((10, _H), lambda i: (0, 0)),
            pl.BlockSpec((_H, _H), lambda i: (0, 0)),
            pl.BlockSpec((_H, _H), lambda i: (0, 0)),
            pl.BlockSpec((_H, _H), lambda i: (0, 0)),
        ],
        out_specs=[
            pl.BlockSpec((_BA, _H), lambda i: (i, 0)),
            pl.BlockSpec((_BA, _H), lambda i: (i, 0)),
            pl.BlockSpec((_BA, _H), lambda i: (i, 0)),
            pl.BlockSpec((_BA, _H), lambda i: (i, 0)),
            pl.BlockSpec((_BA, 1), lambda i: (i, 0)),
            pl.BlockSpec((_BA, 1), lambda i: (i, 0)),
        ],
        out_shape=[
            jax.ShapeDtypeStruct((_N, _H), jnp.float32),
            jax.ShapeDtypeStruct((_N, _H), jnp.float32),
            jax.ShapeDtypeStruct((_N, _H), jnp.float32),
            jax.ShapeDtypeStruct((_N, _H), jnp.float32),
            jax.ShapeDtypeStruct((_N, 1), jnp.int32),
            jax.ShapeDtypeStruct((_N, 1), jnp.int32),
        ],
    )(physics_x, structural_x, W_struct, b_struct.reshape(1, _H),
      gamma.reshape(1, _H), beta.reshape(1, _H), W_phys, Wq, Wk, Wv)

    gk_row = gk.reshape(1, _N)
    nuc_row = nuc_col.reshape(1, _N)

    ctx = pl.pallas_call(
        _attn,
        grid=(_N // _BQ,),
        in_specs=[
            pl.BlockSpec((_BQ, 1), lambda i: (i, 0)),
            pl.BlockSpec((1, _N), lambda i: (0, 0)),
            pl.BlockSpec((_BQ, 1), lambda i: (i, 0)),
            pl.BlockSpec((1, _N), lambda i: (0, 0)),
            pl.BlockSpec((_BQ, _H), lambda i: (i, 0)),
            pl.BlockSpec((_N, _H), lambda i: (0, 0)),
            pl.BlockSpec((_N, _H), lambda i: (0, 0)),
        ],
        out_specs=pl.BlockSpec((_BQ, _H), lambda i: (i, 0)),
        out_shape=jax.ShapeDtypeStruct((_N, _H), jnp.float32),
    )(fq, gk_row, nuc_col, nuc_row, q, k, v)

    out = pl.pallas_call(
        _out_proj,
        grid=(_N // _BA,),
        in_specs=[
            pl.BlockSpec((_BA, _H), lambda i: (i, 0)),
            pl.BlockSpec((_BA, _H), lambda i: (i, 0)),
            pl.BlockSpec((_H, _H), lambda i: (0, 0)),
        ],
        out_specs=pl.BlockSpec((_BA, _H), lambda i: (i, 0)),
        out_shape=jax.ShapeDtypeStruct((_N, _H), jnp.float32),
    )(h, ctx, Wo)
    return out
